# Initial kernel scaffold; baseline (speedup 1.0000x reference)
import jax
import jax.numpy as jnp
from jax.experimental import pallas as pl


def kernel(x, edge_index, W1, b1, W2, b2, temp):
    raise NotImplementedError("write your pallas kernel here")


# speedup vs baseline: 23.3177x; 23.3177x over previous
import functools

import jax
import jax.numpy as jnp
from jax import lax
from jax.experimental import pallas as pl
from jax.experimental.pallas import tpu as pltpu
from jax.experimental.pallas import tpu_sc as plsc

N = 10000
F_IN = 256
HID = 512
C = 64
K = 10
E = 160000

NC = 2
NT = 16
CH = C // NC
WSZ = 128
NWIN = 79
EPT = NWIN * WSZ
E_PAD = NT * EPT
N_PAD = 10240
RPT = N_PAD // NT
LASTR = N - (NT - 1) * RPT
NPADROWS = N_PAD - N


def _mlp_body(x_ref, w1_ref, b1_ref, w2_ref, b2_ref, o_ref):
    h = jnp.dot(x_ref[...], w1_ref[...], preferred_element_type=jnp.float32)
    h = jnp.maximum(h + b1_ref[...], 0.0)
    res = jnp.dot(h, w2_ref[...], preferred_element_type=jnp.float32) + b2_ref[...]
    o_ref[0] = res[:, :CH]
    o_ref[1] = res[:, CH:]


def _mlp(x, W1, b1, W2, b2):
    blk = N // 10
    return pl.pallas_call(
        _mlp_body,
        grid=(10,),
        in_specs=[
            pl.BlockSpec((blk, F_IN), lambda i: (i, 0)),
            pl.BlockSpec((F_IN, HID), lambda i: (0, 0)),
            pl.BlockSpec((1, HID), lambda i: (0, 0)),
            pl.BlockSpec((HID, C), lambda i: (0, 0)),
            pl.BlockSpec((1, C), lambda i: (0, 0)),
        ],
        out_specs=pl.BlockSpec((NC, blk, CH), lambda i: (0, i, 0)),
        out_shape=jax.ShapeDtypeStruct((NC, N, CH), jnp.float32),
    )(x, W1, b1.reshape(1, HID), W2, b2.reshape(1, C))


def _lsm_body(h_ref, o_ref):
    v = jnp.concatenate([h_ref[0], h_ref[1]], axis=1)
    m = jnp.max(v, axis=1, keepdims=True)
    o_ref[...] = v - m - jnp.log(jnp.sum(jnp.exp(v - m), axis=1, keepdims=True))


def _log_softmax(h):
    blk = N // 10
    return pl.pallas_call(
        _lsm_body,
        grid=(10,),
        in_specs=[pl.BlockSpec((NC, blk, CH), lambda i: (0, i, 0))],
        out_specs=pl.BlockSpec((blk, C), lambda i: (i, 0)),
        out_shape=jax.ShapeDtypeStruct((N, C), jnp.float32),
    )(h)


def _prop_body(h_hbm, src_hbm, dst_hbm, temp_hbm, out_hbm,
               src_ids, dst_ids, rows, ub, acc, r_loc, d2_loc, sq_loc,
               ones_b, temp_loc, v_sp, t_sp, deg_sp):
    f32 = jnp.float32

    if True:
        c = lax.axis_index("c")
        s = lax.axis_index("s")
        row0 = s * RPT

        pltpu.sync_copy(src_hbm.at[s], src_ids)
        pltpu.sync_copy(dst_hbm.at[s], dst_ids)
        pltpu.sync_copy(temp_hbm, temp_loc)

        def zrow(i, _):
            d2_loc[pl.ds(i * 16, 16)] = jnp.zeros((16,), f32)
            return _
        lax.fori_loop(0, RPT // 16, zrow, None)
        for i in range(WSZ // 16):
            ones_b[pl.ds(i * 16, 16)] = jnp.full((16,), 1.0, f32)
        pltpu.sync_copy(d2_loc, deg_sp.at[pl.ds(row0, RPT)])
        plsc.subcore_barrier()

        def degw(w, _):
            pltpu.sync_copy(ones_b, deg_sp.at[dst_ids.at[w]], add=True)
            return _
        lax.fori_loop(0, NWIN, degw, None)
        plsc.subcore_barrier()

        pltpu.sync_copy(deg_sp.at[pl.ds(row0, RPT)], r_loc)

        def rsq(i, _):
            sl = pl.ds(i * 16, 16)
            d = r_loc[sl] + 1.0
            y = lax.bitcast_convert_type(d, jnp.int32)
            y = jnp.int32(0x5F3759DF) - lax.shift_right_logical(y, 1)
            g = lax.bitcast_convert_type(y, f32)
            for _n in range(4):
                g = g * (1.5 - 0.5 * d * g * g)
            r_loc[sl] = g
            d2_loc[sl] = 1.0 / d
            sq_loc[sl] = d * g
            return _
        lax.fori_loop(0, RPT // 16, rsq, None)

        @pl.when(s < NT - 1)
        def _():
            pltpu.sync_copy(h_hbm.at[c, pl.ds(row0, RPT)], ub)

        @pl.when(s == NT - 1)
        def _():
            def zpad(i, _):
                ub[LASTR + i, pl.ds(0, 16)] = jnp.zeros((16,), f32)
                ub[LASTR + i, pl.ds(16, 16)] = jnp.zeros((16,), f32)
                return _
            lax.fori_loop(0, RPT - LASTR, zpad, None)
            pltpu.sync_copy(
                h_hbm.at[c, pl.ds(row0, LASTR)],
                ub.at[pl.ds(0, LASTR)],
            )

        tv = temp_loc[pl.ds(0, 16)]

        t0 = tv[0]

        def initr(i, _):
            rv = r_loc[pl.ds(i * 16, 16)]
            qv = sq_loc[pl.ds(i * 16, 16)]
            for j in range(16):
                for half in range(2):
                    sl = pl.ds(half * 16, 16)
                    hv = ub[i * 16 + j, sl]
                    acc[i * 16 + j, sl] = (t0 * qv[j]) * hv
                    ub[i * 16 + j, sl] = rv[j] * hv
            return _
        lax.fori_loop(0, RPT // 16, initr, None)
        pltpu.sync_copy(ub, v_sp.at[pl.ds(row0, RPT)])
        pltpu.sync_copy(ub, t_sp.at[pl.ds(row0, RPT)])
        plsc.subcore_barrier()

        for k in range(1, K + 1):
            def w_body(w, _):
                pltpu.sync_copy(v_sp.at[src_ids.at[w]], rows)
                pltpu.sync_copy(rows, t_sp.at[dst_ids.at[w]], add=True)
                return _
            lax.fori_loop(0, NWIN, w_body, None)
            plsc.subcore_barrier()

            pltpu.sync_copy(t_sp.at[pl.ds(row0, RPT)], ub)
            tk = tv[k]

            def row_body(i, _):
                dv = d2_loc[pl.ds(i * 16, 16)]
                for j in range(16):
                    for half in range(2):
                        sl = pl.ds(half * 16, 16)
                        u = ub[i * 16 + j, sl]
                        acc[i * 16 + j, sl] = acc[i * 16 + j, sl] + tk * u
                        ub[i * 16 + j, sl] = dv[j] * u
                return _
            lax.fori_loop(0, RPT // 16, row_body, None)
            pltpu.sync_copy(ub, v_sp.at[pl.ds(row0, RPT)])
            pltpu.sync_copy(ub, t_sp.at[pl.ds(row0, RPT)])
            plsc.subcore_barrier()

        def outr(i, _):
            rv = r_loc[pl.ds(i * 16, 16)]
            for j in range(16):
                for half in range(2):
                    sl = pl.ds(half * 16, 16)
                    ub[i * 16 + j, sl] = rv[j] * acc[i * 16 + j, sl]
            return _
        lax.fori_loop(0, RPT // 16, outr, None)

        @pl.when(s < NT - 1)
        def _():
            pltpu.sync_copy(ub, out_hbm.at[c, pl.ds(row0, RPT)])

        @pl.when(s == NT - 1)
        def _():
            pltpu.sync_copy(
                ub.at[pl.ds(0, LASTR)],
                out_hbm.at[c, pl.ds(row0, LASTR)],
            )

@functools.partial(
    pl.kernel,
    out_type=jax.ShapeDtypeStruct((NC, N, CH), jnp.float32),
    mesh=plsc.VectorSubcoreMesh(core_axis_name="c", subcore_axis_name="s"),
    compiler_params=pltpu.CompilerParams(use_tc_tiling_on_sc=False),
    scratch_types=[
        pltpu.VMEM((NWIN, WSZ), jnp.int32),
        pltpu.VMEM((NWIN, WSZ), jnp.int32),
        pltpu.VMEM((WSZ, CH), jnp.float32),
        pltpu.VMEM((RPT, CH), jnp.float32),
        pltpu.VMEM((RPT, CH), jnp.float32),
        pltpu.VMEM((RPT,), jnp.float32),
        pltpu.VMEM((RPT,), jnp.float32),
        pltpu.VMEM((RPT,), jnp.float32),
        pltpu.VMEM((WSZ,), jnp.float32),
        pltpu.VMEM((16,), jnp.float32),
        pltpu.VMEM_SHARED((N_PAD, CH), jnp.float32),
        pltpu.VMEM_SHARED((N_PAD, CH), jnp.float32),
        pltpu.VMEM_SHARED((N_PAD,), jnp.float32),
    ],
)
def _propagate(h_hbm, src_hbm, dst_hbm, temp_hbm, out_hbm, *scratch):
    _prop_body(h_hbm, src_hbm, dst_hbm, temp_hbm, out_hbm, *scratch)


def kernel(x, edge_index, W1, b1, W2, b2, temp):
    h = _mlp(x, W1, b1, W2, b2)
    pad = (jnp.arange(E_PAD - E, dtype=jnp.int32) % NPADROWS) + N
    srcp = jnp.concatenate([edge_index[0], pad]).reshape(NT, NWIN, WSZ)
    dstp = jnp.concatenate([edge_index[1], pad]).reshape(NT, NWIN, WSZ)
    temp_pad = jnp.pad(temp, (0, 16 - (K + 1)))
    hidden = _propagate(h, srcp, dstp, temp_pad)
    return _log_softmax(hidden)


# speedup vs baseline: 27.9111x; 1.1970x over previous
import functools

import jax
import jax.numpy as jnp
from jax import lax
from jax.experimental import pallas as pl
from jax.experimental.pallas import tpu as pltpu
from jax.experimental.pallas import tpu_sc as plsc

N = 10000
F_IN = 256
HID = 512
C = 64
K = 10
E = 160000

NC = 2
NT = 16
CH = C // NC
WSZ = 128
NWIN = 80
EPT = NWIN * WSZ
E_PAD = NT * EPT
N_PAD = 10240
RPT = N_PAD // NT
LASTR = N - (NT - 1) * RPT
NPADROWS = N_PAD - N


def _mlp_body(x_ref, w1_ref, b1_ref, w2_ref, b2_ref, o_ref):
    h = jnp.dot(x_ref[...], w1_ref[...], preferred_element_type=jnp.float32)
    h = jnp.maximum(h + b1_ref[...], 0.0)
    res = jnp.dot(h, w2_ref[...], preferred_element_type=jnp.float32) + b2_ref[...]
    o_ref[0] = res[:, :CH]
    o_ref[1] = res[:, CH:]


def _mlp(x, W1, b1, W2, b2):
    blk = N // 10
    return pl.pallas_call(
        _mlp_body,
        grid=(10,),
        in_specs=[
            pl.BlockSpec((blk, F_IN), lambda i: (i, 0)),
            pl.BlockSpec((F_IN, HID), lambda i: (0, 0)),
            pl.BlockSpec((1, HID), lambda i: (0, 0)),
            pl.BlockSpec((HID, C), lambda i: (0, 0)),
            pl.BlockSpec((1, C), lambda i: (0, 0)),
        ],
        out_specs=pl.BlockSpec((NC, blk, CH), lambda i: (0, i, 0)),
        out_shape=jax.ShapeDtypeStruct((NC, N, CH), jnp.float32),
    )(x, W1, b1.reshape(1, HID), W2, b2.reshape(1, C))


def _lsm_body(h_ref, o_ref):
    v = jnp.concatenate([h_ref[0], h_ref[1]], axis=1)
    m = jnp.max(v, axis=1, keepdims=True)
    o_ref[...] = v - m - jnp.log(jnp.sum(jnp.exp(v - m), axis=1, keepdims=True))


def _log_softmax(h):
    blk = N // 10
    return pl.pallas_call(
        _lsm_body,
        grid=(10,),
        in_specs=[pl.BlockSpec((NC, blk, CH), lambda i: (0, i, 0))],
        out_specs=pl.BlockSpec((blk, C), lambda i: (i, 0)),
        out_shape=jax.ShapeDtypeStruct((N, C), jnp.float32),
    )(h)


def _prop_body(h_hbm, src_hbm, dst_hbm, temp_hbm, out_hbm,
               src_ids, dst_ids, rows_a, rows_b, ub, acc, r_loc, d2_loc, sq_loc,
               ones_b, temp_loc, v_sp, t_sp, deg_sp, semga, semgb, semsa, semsb):
    f32 = jnp.float32

    if True:
        c = lax.axis_index("c")
        s = lax.axis_index("s")
        row0 = s * RPT

        pltpu.sync_copy(src_hbm.at[s], src_ids)
        pltpu.sync_copy(dst_hbm.at[s], dst_ids)
        pltpu.sync_copy(temp_hbm, temp_loc)

        def zrow(i, _):
            d2_loc[pl.ds(i * 16, 16)] = jnp.zeros((16,), f32)
            return _
        lax.fori_loop(0, RPT // 16, zrow, None)
        for i in range(WSZ // 16):
            ones_b[pl.ds(i * 16, 16)] = jnp.full((16,), 1.0, f32)
        pltpu.sync_copy(d2_loc, deg_sp.at[pl.ds(row0, RPT)])
        plsc.subcore_barrier()

        def degw(w, _):
            pltpu.sync_copy(ones_b, deg_sp.at[dst_ids.at[w]], add=True)
            return _
        lax.fori_loop(0, NWIN, degw, None)
        plsc.subcore_barrier()

        pltpu.sync_copy(deg_sp.at[pl.ds(row0, RPT)], r_loc)

        def rsq(i, _):
            sl = pl.ds(i * 16, 16)
            d = r_loc[sl] + 1.0
            y = lax.bitcast_convert_type(d, jnp.int32)
            y = jnp.int32(0x5F3759DF) - lax.shift_right_logical(y, 1)
            g = lax.bitcast_convert_type(y, f32)
            for _n in range(4):
                g = g * (1.5 - 0.5 * d * g * g)
            r_loc[sl] = g
            d2_loc[sl] = 1.0 / d
            sq_loc[sl] = d * g
            return _
        lax.fori_loop(0, RPT // 16, rsq, None)

        @pl.when(s < NT - 1)
        def _():
            pltpu.sync_copy(h_hbm.at[c, pl.ds(row0, RPT)], ub)

        @pl.when(s == NT - 1)
        def _():
            def zpad(i, _):
                ub[LASTR + i, pl.ds(0, 16)] = jnp.zeros((16,), f32)
                ub[LASTR + i, pl.ds(16, 16)] = jnp.zeros((16,), f32)
                return _
            lax.fori_loop(0, RPT - LASTR, zpad, None)
            pltpu.sync_copy(
                h_hbm.at[c, pl.ds(row0, LASTR)],
                ub.at[pl.ds(0, LASTR)],
            )

        tv = temp_loc[pl.ds(0, 16)]

        t0 = tv[0]

        def initr(i, _):
            rv = r_loc[pl.ds(i * 16, 16)]
            qv = sq_loc[pl.ds(i * 16, 16)]
            for j in range(16):
                for half in range(2):
                    sl = pl.ds(half * 16, 16)
                    hv = ub[i * 16 + j, sl]
                    acc[i * 16 + j, sl] = (t0 * qv[j]) * hv
                    ub[i * 16 + j, sl] = rv[j] * hv
            return _
        lax.fori_loop(0, RPT // 16, initr, None)
        pltpu.sync_copy(ub, v_sp.at[pl.ds(row0, RPT)])
        pltpu.sync_copy(ub, t_sp.at[pl.ds(row0, RPT)])
        plsc.subcore_barrier()

        NP = NWIN // 2
        for k in range(1, K + 1):
            pltpu.async_copy(v_sp.at[src_ids.at[0]], rows_a, semga)
            pltpu.async_copy(v_sp.at[src_ids.at[1]], rows_b, semgb)

            def pair(i, _):
                w0 = 2 * i
                w1 = w0 + 1
                pltpu.make_async_copy(v_sp.at[src_ids.at[w0]], rows_a, semga).wait()
                pltpu.async_copy(rows_a, t_sp.at[dst_ids.at[w0]], semsa, add=True)
                pltpu.make_async_copy(v_sp.at[src_ids.at[w1]], rows_b, semgb).wait()
                pltpu.async_copy(rows_b, t_sp.at[dst_ids.at[w1]], semsb, add=True)

                @pl.when(i < NP - 1)
                def _():
                    pltpu.make_async_copy(
                        rows_a, t_sp.at[dst_ids.at[w0]], semsa).wait()
                    pltpu.async_copy(v_sp.at[src_ids.at[w0 + 2]], rows_a, semga)
                    pltpu.make_async_copy(
                        rows_b, t_sp.at[dst_ids.at[w1]], semsb).wait()
                    pltpu.async_copy(v_sp.at[src_ids.at[w1 + 2]], rows_b, semgb)
                return _
            lax.fori_loop(0, NP, pair, None)
            pltpu.make_async_copy(rows_a, t_sp.at[dst_ids.at[0]], semsa).wait()
            pltpu.make_async_copy(rows_b, t_sp.at[dst_ids.at[1]], semsb).wait()
            plsc.subcore_barrier()

            pltpu.sync_copy(t_sp.at[pl.ds(row0, RPT)], ub)
            tk = tv[k]

            def row_body(i, _):
                dv = d2_loc[pl.ds(i * 16, 16)]
                for j in range(16):
                    for half in range(2):
                        sl = pl.ds(half * 16, 16)
                        u = ub[i * 16 + j, sl]
                        acc[i * 16 + j, sl] = acc[i * 16 + j, sl] + tk * u
                        ub[i * 16 + j, sl] = dv[j] * u
                return _
            lax.fori_loop(0, RPT // 16, row_body, None)
            pltpu.sync_copy(ub, v_sp.at[pl.ds(row0, RPT)])
            pltpu.sync_copy(ub, t_sp.at[pl.ds(row0, RPT)])
            plsc.subcore_barrier()

        def outr(i, _):
            rv = r_loc[pl.ds(i * 16, 16)]
            for j in range(16):
                for half in range(2):
                    sl = pl.ds(half * 16, 16)
                    ub[i * 16 + j, sl] = rv[j] * acc[i * 16 + j, sl]
            return _
        lax.fori_loop(0, RPT // 16, outr, None)

        @pl.when(s < NT - 1)
        def _():
            pltpu.sync_copy(ub, out_hbm.at[c, pl.ds(row0, RPT)])

        @pl.when(s == NT - 1)
        def _():
            pltpu.sync_copy(
                ub.at[pl.ds(0, LASTR)],
                out_hbm.at[c, pl.ds(row0, LASTR)],
            )

@functools.partial(
    pl.kernel,
    out_type=jax.ShapeDtypeStruct((NC, N, CH), jnp.float32),
    mesh=plsc.VectorSubcoreMesh(core_axis_name="c", subcore_axis_name="s"),
    compiler_params=pltpu.CompilerParams(use_tc_tiling_on_sc=False),
    scratch_types=[
        pltpu.VMEM((NWIN, WSZ), jnp.int32),
        pltpu.VMEM((NWIN, WSZ), jnp.int32),
        pltpu.VMEM((WSZ, CH), jnp.float32),
        pltpu.VMEM((WSZ, CH), jnp.float32),
        pltpu.VMEM((RPT, CH), jnp.float32),
        pltpu.VMEM((RPT, CH), jnp.float32),
        pltpu.VMEM((RPT,), jnp.float32),
        pltpu.VMEM((RPT,), jnp.float32),
        pltpu.VMEM((RPT,), jnp.float32),
        pltpu.VMEM((WSZ,), jnp.float32),
        pltpu.VMEM((16,), jnp.float32),
        pltpu.VMEM_SHARED((N_PAD, CH), jnp.float32),
        pltpu.VMEM_SHARED((N_PAD, CH), jnp.float32),
        pltpu.VMEM_SHARED((N_PAD,), jnp.float32),
        pltpu.SemaphoreType.DMA,
        pltpu.SemaphoreType.DMA,
        pltpu.SemaphoreType.DMA,
        pltpu.SemaphoreType.DMA,
    ],
)
def _propagate(h_hbm, src_hbm, dst_hbm, temp_hbm, out_hbm, *scratch):
    _prop_body(h_hbm, src_hbm, dst_hbm, temp_hbm, out_hbm, *scratch)


def kernel(x, edge_index, W1, b1, W2, b2, temp):
    h = _mlp(x, W1, b1, W2, b2)
    pad = (jnp.arange(E_PAD - E, dtype=jnp.int32) % NPADROWS) + N
    srcp = jnp.concatenate([edge_index[0], pad]).reshape(NT, NWIN, WSZ)
    dstp = jnp.concatenate([edge_index[1], pad]).reshape(NT, NWIN, WSZ)
    temp_pad = jnp.pad(temp, (0, 16 - (K + 1)))
    hidden = _propagate(h, srcp, dstp, temp_pad)
    return _log_softmax(hidden)
